# baseline (speedup 1.0000x reference)
import functools

import jax
import jax.numpy as jnp
from jax import lax
from jax.experimental import pallas as pl
from jax.experimental.pallas import tpu as pltpu
from jax.experimental.pallas import tpu_sc as plsc

_PADDING_IDX = 1
_BATCH = 4
_SEQ = 8192
_D = 1024
_ROWS = _BATCH * _SEQ
_NC = 2
_NS = 16
_NW = _NC * _NS
_SQ = _SEQ // _NW
_L = 16
_NGB = _SQ // _L
_NWORDS = _ROWS // _L

_B = 2048
_CHW = _B * _D
_NCHT = _SEQ // _B
_WPB = _SEQ // _L
_WPC = _B // _L

_mesh = plsc.VectorSubcoreMesh(core_axis_name="c", subcore_axis_name="s")

_dn = lax.GatherDimensionNumbers(
    offset_dims=(), collapsed_slice_dims=(0,), start_index_map=(0,))


def _take16(v, idx):
    return lax.gather(v, idx.reshape(_L, 1), _dn, slice_sizes=(1,),
                      mode=lax.GatherScatterMode.PROMISE_IN_BOUNDS)


@functools.partial(
    pl.kernel,
    mesh=_mesh,
    out_type=jax.ShapeDtypeStruct((_NWORDS,), jnp.int32),
    scratch_types=[
        pltpu.VMEM((_BATCH * _SQ,), jnp.int32),
        pltpu.VMEM((_L,), jnp.int32),
    ],
)
def _sc_plan(x_hbm, plan_hbm, xb_v, wordv):
    wid = lax.axis_index("s") * _NC + lax.axis_index("c")
    s0 = wid * _SQ
    lanes = lax.iota(jnp.int32, _L)

    for b in range(_BATCH):
        pltpu.sync_copy(x_hbm.at[pl.ds(b * _SEQ + s0, _SQ)],
                        xb_v.at[pl.ds(b * _SQ, _SQ)])

    for b in range(_BATCH):
        wvec = jnp.zeros((_L,), jnp.int32)
        for i in range(_NGB):
            xv = xb_v[pl.ds(b * _SQ + i * _L, _L)]
            bits = jnp.where(xv == _PADDING_IDX,
                             lax.shift_left(jnp.int32(1), lanes), 0)
            for sh in (1, 2, 4, 8):
                bits = lax.bitwise_or(
                    bits, _take16(bits, lax.bitwise_xor(lanes, sh)))
            wvec = jnp.where(lanes == i, bits, wvec)
        wordv[pl.ds(0, _L)] = wvec
        pltpu.sync_copy(wordv,
                        plan_hbm.at[pl.ds(b * _WPB + wid * _NGB, _NGB)])


def _tc_body(w_ref, plan_ref, o_ref, s0_ref, s1_ref, plan_s, padv,
             g0, g1, w0, w1, auxsem):
    scr = (s0_ref, s1_ref)
    gsem = (g0, g1)
    wsem = (w0, w1)

    def gcopy(k, buf):
        return pltpu.make_async_copy(
            w_ref.at[pl.ds((k * _B + _PADDING_IDX + 1) * _D, _CHW)],
            scr[buf], gsem[buf])

    def wcopy(k, b, buf):
        return pltpu.make_async_copy(
            scr[buf], o_ref.at[pl.ds((b * _SEQ + k * _B) * _D, _CHW)],
            wsem[buf])

    def fixups(k):
        def scan(i, carry):
            b = lax.shift_right_logical(i, 7)
            j = lax.bitwise_and(i, _WPC - 1)
            g = b * _WPB + k * _WPC + j
            word = plan_s[g]

            @pl.when(word != 0)
            def _():
                for t in range(_L):
                    @pl.when(lax.bitwise_and(
                        lax.shift_right_logical(word, t), 1) != 0)
                    def _():
                        cp = pltpu.make_async_copy(
                            padv, o_ref.at[pl.ds((g * _L + t) * _D, _D)],
                            auxsem)
                        cp.start()
                        cp.wait()

            return carry

        lax.fori_loop(0, _BATCH * _WPC, scan, 0)

    plan_cp = pltpu.make_async_copy(plan_ref, plan_s, auxsem)
    pad_cp = pltpu.make_async_copy(
        w_ref.at[pl.ds(_PADDING_IDX * _D, _D)], padv, auxsem)
    plan_cp.start()
    pad_cp.start()
    gcopy(0, 0).start()
    plan_cp.wait()
    pad_cp.wait()

    for k in range(_NCHT):
        buf = k % 2
        gcopy(k, buf).wait()
        for b in range(_BATCH):
            wcopy(k, b, buf).start()
        if k + 1 < _NCHT:
            if k >= 1:
                for b in range(_BATCH):
                    wcopy(k - 1, b, (k - 1) % 2).wait()
                fixups(k - 1)
            gcopy(k + 1, (k + 1) % 2).start()
    for k in (_NCHT - 2, _NCHT - 1):
        for b in range(_BATCH):
            wcopy(k, b, k % 2).wait()
        fixups(k)


_tc_bulk = pl.pallas_call(
    _tc_body,
    out_shape=jax.ShapeDtypeStruct((_ROWS * _D,), jnp.float32),
    in_specs=[pl.BlockSpec(memory_space=pl.ANY),
              pl.BlockSpec(memory_space=pl.ANY)],
    out_specs=pl.BlockSpec(memory_space=pl.ANY),
    scratch_shapes=[
        pltpu.VMEM((_CHW,), jnp.float32),
        pltpu.VMEM((_CHW,), jnp.float32),
        pltpu.SMEM((_NWORDS,), jnp.int32),
        pltpu.VMEM((_D,), jnp.float32),
        pltpu.SemaphoreType.DMA,
        pltpu.SemaphoreType.DMA,
        pltpu.SemaphoreType.DMA,
        pltpu.SemaphoreType.DMA,
        pltpu.SemaphoreType.DMA,
    ],
)


def kernel(x, weights):
    bsz, seq_len = x.shape
    xf = x.reshape(bsz * seq_len).astype(jnp.int32)
    plan = _sc_plan(xf)
    out = _tc_bulk(weights.reshape(-1), plan)
    return lax.stop_gradient(out.reshape(bsz, seq_len, _D))
